# baseline (speedup 1.0000x reference)
import functools

import jax
import jax.numpy as jnp
from jax import lax
from jax.experimental import pallas as pl
from jax.experimental.pallas import tpu as pltpu
from jax.experimental.pallas import tpu_sc as plsc

_N = 576 * 576
_H = 32
_ROWS = 2209
_NC, _NS, _L = 2, 16, 16
_NW = _NC * _NS
_PER_W = _N // _NW
_SUB = 384
_NSUB = _PER_W // _SUB
_NV = _SUB // _L


def _make_kernel():
    mesh = plsc.VectorSubcoreMesh(core_axis_name="c", subcore_axis_name="s")

    @functools.partial(
        pl.kernel,
        mesh=mesh,
        out_type=jax.ShapeDtypeStruct((_H, _N), jnp.float32),
        scratch_types=[
            pltpu.VMEM((_ROWS * _H,), jnp.float32),
            pltpu.VMEM((_PER_W,), jnp.int32),
            pltpu.VMEM((2, _H, _SUB), jnp.float32),
            pltpu.SemaphoreType.DMA,
            pltpu.SemaphoreType.DMA,
            pltpu.SemaphoreType.DMA,
        ],
        compiler_params=pltpu.CompilerParams(
            use_tc_tiling_on_sc=False,
            needs_layout_passes=False,
            disable_bounds_checks=True,
            disable_semaphore_checks=True,
        ),
    )
    def k(table_hbm, idx_hbm, out_hbm, table_v, idx_v, out_v, sem_in, sem0, sem1):
        wid = lax.axis_index("s") * _NC + lax.axis_index("c")
        base = wid * _PER_W

        cp_t = pltpu.make_async_copy(table_hbm, table_v, sem_in)
        cp_t.start()
        cp_i = pltpu.make_async_copy(idx_hbm.at[pl.ds(base, _PER_W)], idx_v, sem_in)
        cp_i.start()
        cp_t.wait()
        cp_i.wait()

        sems = (sem0, sem1)

        def gather_round(s, buf):
            off = s * _SUB

            @plsc.parallel_loop(0, _NV)
            def _(v):
                iv = idx_v[pl.ds(off + v * _L, _L)]

                @plsc.parallel_loop(0, _H, unroll=8)
                def _(h):
                    out_v[buf, h, pl.ds(v * _L, _L)] = plsc.load_gather(
                        table_v, [iv + h * _ROWS]
                    )

            pltpu.make_async_copy(
                out_v.at[buf], out_hbm.at[:, pl.ds(base + off, _SUB)], sems[buf]
            ).start()

        def wait_round(buf):
            pltpu.make_async_copy(
                out_v.at[buf], out_hbm.at[:, pl.ds(0, _SUB)], sems[buf]
            ).wait()

        def round_with_parity(s):
            @pl.when(s % 2 == 0)
            def _():
                gather_round(s, 0)

            @pl.when(s % 2 == 1)
            def _():
                gather_round(s, 1)

        def body(s, carry):
            @pl.when(s >= 2)
            def _():
                @pl.when(s % 2 == 0)
                def _():
                    wait_round(0)

                @pl.when(s % 2 == 1)
                def _():
                    wait_round(1)

            round_with_parity(s)
            return carry

        lax.fori_loop(0, _NSUB, body, 0)
        wait_round(1)
        wait_round(0)

    return k


_gather_kernel = _make_kernel()


def kernel(relative_position_bias_table, relative_position_index):
    table_flat = relative_position_bias_table.T.reshape(-1)
    idx_flat = relative_position_index.reshape(-1).astype(jnp.int32)
    out = _gather_kernel(table_flat, idx_flat)
    n0, n1 = relative_position_index.shape
    return out.reshape(_H, n0, n1)


# speedup vs baseline: 1.0226x; 1.0226x over previous
import functools

import jax
import jax.numpy as jnp
from jax import lax
from jax.experimental import pallas as pl
from jax.experimental.pallas import tpu as pltpu
from jax.experimental.pallas import tpu_sc as plsc

_N0 = 576
_N = _N0 * _N0
_H = 32
_ROWS = 2209
_RPAD = 2304
_NC, _NS, _L = 2, 16, 16
_R = 24
_NSUB = _N0 // _R
_CHUNK = _R * _N0
_NV = _N0 // _L


def _make_kernel():
    mesh = plsc.VectorSubcoreMesh(core_axis_name="c", subcore_axis_name="s")

    @functools.partial(
        pl.kernel,
        mesh=mesh,
        out_type=jax.ShapeDtypeStruct((_H * _N0, _N0), jnp.float32),
        scratch_types=[
            pltpu.VMEM((_RPAD,), jnp.float32),
            pltpu.VMEM((2, _CHUNK), jnp.int32),
            pltpu.VMEM((2, _R, _N0), jnp.float32),
            pltpu.SemaphoreType.DMA,
            pltpu.SemaphoreType.DMA,
            pltpu.SemaphoreType.DMA,
            pltpu.SemaphoreType.DMA,
            pltpu.SemaphoreType.DMA,
        ],
        compiler_params=pltpu.CompilerParams(
            use_tc_tiling_on_sc=True, needs_layout_passes=False
        ),
    )
    def k(table_hbm, idx_hbm, out_hbm, trow_v, idx_v, out_v,
          sem_t, sem_i0, sem_i1, sem_o0, sem_o1):
        wid = lax.axis_index("s") * _NC + lax.axis_index("c")
        sem_i = (sem_i0, sem_i1)
        sem_o = (sem_o0, sem_o1)

        cp_t = pltpu.make_async_copy(
            table_hbm.at[pl.ds(wid * _RPAD, _RPAD)], trow_v, sem_t
        )
        cp_t.start()

        def start_idx(s, buf):
            pltpu.make_async_copy(
                idx_hbm.at[pl.ds(s * _CHUNK, _CHUNK)], idx_v.at[buf], sem_i[buf]
            ).start()

        def wait_idx(buf):
            pltpu.make_async_copy(
                idx_hbm.at[pl.ds(0, _CHUNK)], idx_v.at[buf], sem_i[buf]
            ).wait()

        def wait_out(buf):
            pltpu.make_async_copy(
                out_v.at[buf], out_hbm.at[pl.ds(0, _R), :], sem_o[buf]
            ).wait()

        start_idx(0, 0)
        cp_t.wait()
        row0 = wid * _N0

        def gather_round(s, buf):
            @pl.when(s < _NSUB - 1)
            def _():
                start_idx(s + 1, 1 - buf)

            wait_idx(buf)

            @pl.when(s >= 2)
            def _():
                wait_out(buf)

            @plsc.parallel_loop(0, _R)
            def _(r):
                @plsc.parallel_loop(0, _NV, unroll=8)
                def _(c):
                    iv = idx_v[buf, pl.ds(r * _N0 + c * _L, _L)]
                    out_v[buf, r, pl.ds(c * _L, _L)] = plsc.load_gather(
                        trow_v, [iv]
                    )

            pltpu.make_async_copy(
                out_v.at[buf],
                out_hbm.at[pl.ds(row0 + s * _R, _R), :],
                sem_o[buf],
            ).start()

        def body(s, carry):
            @pl.when(s % 2 == 0)
            def _():
                gather_round(s, 0)

            @pl.when(s % 2 == 1)
            def _():
                gather_round(s, 1)

            return carry

        lax.fori_loop(0, _NSUB, body, 0)
        wait_out(0)
        wait_out(1)

    return k


_gather_kernel = _make_kernel()


def kernel(relative_position_bias_table, relative_position_index):
    table_rows = jnp.pad(
        relative_position_bias_table.T, ((0, 0), (0, _RPAD - _ROWS))
    )
    table_flat = table_rows.reshape(-1)
    idx_flat = relative_position_index.reshape(-1).astype(jnp.int32)
    out = _gather_kernel(table_flat, idx_flat)
    n0, n1 = relative_position_index.shape
    return out.reshape(_H, n0, n1)


# speedup vs baseline: 1.8907x; 1.8489x over previous
import functools

import jax
import jax.numpy as jnp
from jax import lax
from jax.experimental import pallas as pl
from jax.experimental.pallas import tpu as pltpu
from jax.experimental.pallas import tpu_sc as plsc

_N0 = 576
_N = _N0 * _N0
_H = 32
_ROWS = 2209
_RPAD = 2304
_NC, _NS, _L = 2, 16, 16
_HG = 4
_NQ = 4
_QROWS = _N0 // _NQ
_R = 8
_NSUB = _QROWS // _R
_CHUNK = _R * _N0
_NV = _N0 // _L


def _make_kernel():
    mesh = plsc.VectorSubcoreMesh(core_axis_name="c", subcore_axis_name="s")

    @functools.partial(
        pl.kernel,
        mesh=mesh,
        out_type=jax.ShapeDtypeStruct((_H * _N0, _N0), jnp.float32),
        scratch_types=[
            pltpu.VMEM((_HG * _RPAD,), jnp.float32),
            pltpu.VMEM((2, _CHUNK), jnp.int32),
            pltpu.VMEM((2, _HG, _R, _N0), jnp.float32),
            pltpu.SemaphoreType.DMA,
            pltpu.SemaphoreType.DMA,
            pltpu.SemaphoreType.DMA,
            pltpu.SemaphoreType.DMA,
            pltpu.SemaphoreType.DMA,
        ],
        compiler_params=pltpu.CompilerParams(
            use_tc_tiling_on_sc=True, needs_layout_passes=False
        ),
    )
    def k(table_hbm, idx_hbm, out_hbm, trows_v, idx_v, out_v,
          sem_t, sem_i0, sem_i1, sem_o0, sem_o1):
        wid = lax.axis_index("s") * _NC + lax.axis_index("c")
        hblk = wid % (_H // _HG)
        quar = wid // (_H // _HG)
        sem_i = (sem_i0, sem_i1)
        sem_o = (sem_o0, sem_o1)

        cp_t = pltpu.make_async_copy(
            table_hbm.at[pl.ds(hblk * _HG * _RPAD, _HG * _RPAD)], trows_v, sem_t
        )
        cp_t.start()

        idx_base = quar * _QROWS * _N0

        def start_idx(s, buf):
            pltpu.make_async_copy(
                idx_hbm.at[pl.ds(idx_base + s * _CHUNK, _CHUNK)],
                idx_v.at[buf],
                sem_i[buf],
            ).start()

        def wait_idx(buf):
            pltpu.make_async_copy(
                idx_hbm.at[pl.ds(0, _CHUNK)], idx_v.at[buf], sem_i[buf]
            ).wait()

        def wait_out(buf):
            for hl in range(_HG):
                pltpu.make_async_copy(
                    out_v.at[buf, hl], out_hbm.at[pl.ds(0, _R), :], sem_o[buf]
                ).wait()

        start_idx(0, 0)
        cp_t.wait()

        def gather_round(s, buf):
            @pl.when(s < _NSUB - 1)
            def _():
                start_idx(s + 1, 1 - buf)

            wait_idx(buf)

            @pl.when(s >= 2)
            def _():
                wait_out(buf)

            @plsc.parallel_loop(0, _R)
            def _(r):
                @plsc.parallel_loop(0, _NV, unroll=4)
                def _(c):
                    iv = idx_v[buf, pl.ds(r * _N0 + c * _L, _L)]
                    for hl in range(_HG):
                        out_v[buf, hl, r, pl.ds(c * _L, _L)] = plsc.load_gather(
                            trows_v, [iv + hl * _RPAD]
                        )

            for hl in range(_HG):
                row0 = (hblk * _HG + hl) * _N0 + quar * _QROWS + s * _R
                pltpu.make_async_copy(
                    out_v.at[buf, hl],
                    out_hbm.at[pl.ds(row0, _R), :],
                    sem_o[buf],
                ).start()

        def body(s, carry):
            @pl.when(s % 2 == 0)
            def _():
                gather_round(s, 0)

            @pl.when(s % 2 == 1)
            def _():
                gather_round(s, 1)

            return carry

        lax.fori_loop(0, _NSUB, body, 0)
        wait_out(0)
        wait_out(1)

    return k


_gather_kernel = _make_kernel()


def kernel(relative_position_bias_table, relative_position_index):
    table_rows = jnp.pad(
        relative_position_bias_table.T, ((0, 0), (0, _RPAD - _ROWS))
    )
    table_flat = table_rows.reshape(-1)
    idx_flat = relative_position_index.reshape(-1).astype(jnp.int32)
    out = _gather_kernel(table_flat, idx_flat)
    n0, n1 = relative_position_index.shape
    return out.reshape(_H, n0, n1)


# speedup vs baseline: 2.2587x; 1.1946x over previous
import functools

import jax
import jax.numpy as jnp
from jax import lax
from jax.experimental import pallas as pl
from jax.experimental.pallas import tpu as pltpu
from jax.experimental.pallas import tpu_sc as plsc

_N0 = 576
_N = _N0 * _N0
_H = 32
_ROWS = 2209
_RPAD = 2304
_NC, _NS, _L = 2, 16, 16
_HG = 8
_NQ = 8
_QROWS = _N0 // _NQ
_R = 8
_NSUB = _QROWS // _R
_CHUNK = _R * _N0
_NV = _N0 // _L


def _make_kernel():
    mesh = plsc.VectorSubcoreMesh(core_axis_name="c", subcore_axis_name="s")

    @functools.partial(
        pl.kernel,
        mesh=mesh,
        out_type=jax.ShapeDtypeStruct((_H * _N0, _N0), jnp.float32),
        scratch_types=[
            pltpu.VMEM((_HG * _RPAD,), jnp.float32),
            pltpu.VMEM((2, _R, _N0), jnp.int32),
            pltpu.VMEM((2, _HG, _R, _N0), jnp.float32),
            pltpu.SemaphoreType.DMA,
            pltpu.SemaphoreType.DMA,
            pltpu.SemaphoreType.DMA,
            pltpu.SemaphoreType.DMA,
            pltpu.SemaphoreType.DMA,
        ],
        compiler_params=pltpu.CompilerParams(
            use_tc_tiling_on_sc=True, needs_layout_passes=False
        ),
    )
    def k(table_hbm, idx_hbm, out_hbm, trows_v, idx_v, out_v,
          sem_t, sem_i0, sem_i1, sem_o0, sem_o1):
        wid = lax.axis_index("s") * _NC + lax.axis_index("c")
        hblk = wid % (_H // _HG)
        quar = wid // (_H // _HG)
        sem_i = (sem_i0, sem_i1)
        sem_o = (sem_o0, sem_o1)

        cp_t = pltpu.make_async_copy(
            table_hbm.at[pl.ds(hblk * _HG * _RPAD, _HG * _RPAD)], trows_v, sem_t
        )
        cp_t.start()

        idx_row0 = quar * _QROWS

        def start_idx(s, buf):
            pltpu.make_async_copy(
                idx_hbm.at[pl.ds(idx_row0 + s * _R, _R), :],
                idx_v.at[buf],
                sem_i[buf],
            ).start()

        def wait_idx(buf):
            pltpu.make_async_copy(
                idx_hbm.at[pl.ds(0, _R), :], idx_v.at[buf], sem_i[buf]
            ).wait()

        def wait_out(buf):
            for hl in range(_HG):
                pltpu.make_async_copy(
                    out_v.at[buf, hl], out_hbm.at[pl.ds(0, _R), :], sem_o[buf]
                ).wait()

        start_idx(0, 0)
        cp_t.wait()

        def gather_round(s, buf):
            @pl.when(s < _NSUB - 1)
            def _():
                start_idx(s + 1, 1 - buf)

            wait_idx(buf)

            @pl.when(s >= 2)
            def _():
                wait_out(buf)

            @plsc.parallel_loop(0, _R)
            def _(r):
                @plsc.parallel_loop(0, _NV, unroll=2)
                def _(c):
                    iv = idx_v[buf, r, pl.ds(c * _L, _L)]
                    for hl in range(_HG):
                        out_v[buf, hl, r, pl.ds(c * _L, _L)] = plsc.load_gather(
                            trows_v, [iv + hl * _RPAD]
                        )

            for hl in range(_HG):
                row0 = (hblk * _HG + hl) * _N0 + quar * _QROWS + s * _R
                pltpu.make_async_copy(
                    out_v.at[buf, hl],
                    out_hbm.at[pl.ds(row0, _R), :],
                    sem_o[buf],
                ).start()

        def body(s, carry):
            @pl.when(s % 2 == 0)
            def _():
                gather_round(s, 0)

            @pl.when(s % 2 == 1)
            def _():
                gather_round(s, 1)

            return carry

        lax.fori_loop(0, _NSUB, body, 0)
        wait_out(0)
        wait_out(1)

    return k


_gather_kernel = _make_kernel()


def kernel(relative_position_bias_table, relative_position_index):
    table_rows = jnp.pad(
        relative_position_bias_table.T, ((0, 0), (0, _RPAD - _ROWS))
    )
    table_flat = table_rows.reshape(-1)
    idx2d = relative_position_index.astype(jnp.int32)
    out = _gather_kernel(table_flat, idx2d)
    n0, n1 = relative_position_index.shape
    return out.reshape(_H, n0, n1)
